# baseline (speedup 1.0000x reference)
import functools

import jax
import jax.numpy as jnp
from jax import lax
from jax.experimental import pallas as pl
from jax.experimental.pallas import tpu as pltpu, tpu_sc as plsc

N_NODES = 10000
N_EDGES = 320000
NUM_GRAPHS = 128

NC = 2
NS = 16
NW = NC * NS
K = 128
C_T = -(-N_EDGES // (NW * K))
E_PAD = NW * C_T * K
N_PAD = 10112
R = N_PAD // NS
DUMMY_ROW = N_NODES + 1

BM = 1000
GRID = N_NODES // BM


def _make_sc_segsum(D):
  mesh = plsc.VectorSubcoreMesh(core_axis_name="c", subcore_axis_name="s")

  @functools.partial(
      pl.kernel,
      out_type=jax.ShapeDtypeStruct((NC, N_PAD, D), jnp.float32),
      mesh=mesh,
      compiler_params=pltpu.CompilerParams(use_tc_tiling_on_sc=False),
      scratch_types=[
          pltpu.VMEM((C_T, K), jnp.int32),
          pltpu.VMEM((C_T, K), jnp.int32),
          pltpu.VMEM((K, D), jnp.float32),
          pltpu.VMEM_SHARED((N_PAD, D), jnp.float32),
          pltpu.SemaphoreType.DMA,
      ],
  )
  def seg(table_hbm, srcs_hbm, dsts_hbm, zeros_hbm, out_hbm,
          src_v, dst_v, rows_v, acc_sh, sem):
    c = lax.axis_index("c")
    s = lax.axis_index("s")
    wid = s * NC + c

    pltpu.sync_copy(srcs_hbm.at[wid], src_v)
    pltpu.sync_copy(dsts_hbm.at[wid], dst_v)
    pltpu.sync_copy(zeros_hbm, acc_sh.at[pl.ds(s * R, R)])
    plsc.subcore_barrier()

    def body(j, carry):
      pltpu.async_copy(table_hbm.at[src_v.at[j]], rows_v, sem).wait()
      pltpu.sync_copy(rows_v, acc_sh.at[dst_v.at[j]], add=True)
      return carry

    lax.fori_loop(0, C_T, body, 0)
    plsc.subcore_barrier()
    pltpu.sync_copy(acc_sh.at[pl.ds(s * R, R)],
                    out_hbm.at[c, pl.ds(s * R, R)])

  return seg


_sc_segsum64 = _make_sc_segsum(64)
_sc_segsum128 = _make_sc_segsum(128)


def _dot(a, b):
  return lax.dot_general(a, b, (((1,), (0,)), ((), ())),
                         preferred_element_type=jnp.float32)


def _tc1_body(x_ref, ws_ref, bs_ref, wr_ref, wo_ref, b1_ref,
              table_ref, root_ref):
  h0 = _dot(x_ref[...], ws_ref[...]) + bs_ref[...]
  table_ref[...] = _dot(h0, wr_ref[...])
  root_ref[...] = _dot(h0, wo_ref[...]) + b1_ref[...]


def _tc1(x, ws, bs, wr, wo, b1):
  return pl.pallas_call(
      _tc1_body,
      grid=(GRID,),
      in_specs=[
          pl.BlockSpec((BM, 128), lambda i: (i, 0)),
          pl.BlockSpec((128, 128), lambda i: (0, 0)),
          pl.BlockSpec((1, 128), lambda i: (0, 0)),
          pl.BlockSpec((128, 64), lambda i: (0, 0)),
          pl.BlockSpec((128, 64), lambda i: (0, 0)),
          pl.BlockSpec((1, 64), lambda i: (0, 0)),
      ],
      out_specs=[
          pl.BlockSpec((BM, 64), lambda i: (i, 0)),
          pl.BlockSpec((BM, 64), lambda i: (i, 0)),
      ],
      out_shape=[
          jax.ShapeDtypeStruct((N_NODES, 64), jnp.float32),
          jax.ShapeDtypeStruct((N_NODES, 64), jnp.float32),
      ],
  )(x, ws, bs, wr, wo, b1)


def _tc2_body(agg_ref, root1_ref, wo2_ref, b2_ref, table_ref, root2_ref):
  h1 = jnp.maximum(agg_ref[0] + agg_ref[1] + root1_ref[...], 0.0)
  table_ref[...] = h1
  root2_ref[...] = _dot(h1, wo2_ref[...]) + b2_ref[...]


def _tc2(agg, root1, wo2, b2):
  return pl.pallas_call(
      _tc2_body,
      grid=(GRID,),
      in_specs=[
          pl.BlockSpec((NC, BM, 64), lambda i: (0, i, 0)),
          pl.BlockSpec((BM, 64), lambda i: (i, 0)),
          pl.BlockSpec((64, 128), lambda i: (0, 0)),
          pl.BlockSpec((1, 128), lambda i: (0, 0)),
      ],
      out_specs=[
          pl.BlockSpec((BM, 64), lambda i: (i, 0)),
          pl.BlockSpec((BM, 128), lambda i: (i, 0)),
      ],
      out_shape=[
          jax.ShapeDtypeStruct((N_NODES, 64), jnp.float32),
          jax.ShapeDtypeStruct((N_NODES, 128), jnp.float32),
      ],
  )(agg, root1, wo2, b2)


def _tc3_body(agg_ref, wr2_ref, root2_ref, wo3_ref, b3_ref,
              table_ref, root3_ref):
  h2 = jnp.maximum(_dot(agg_ref[0] + agg_ref[1], wr2_ref[...])
                   + root2_ref[...], 0.0)
  table_ref[...] = h2
  root3_ref[...] = _dot(h2, wo3_ref[...]) + b3_ref[...]


def _tc3(agg, wr2, root2, wo3, b3):
  return pl.pallas_call(
      _tc3_body,
      grid=(GRID,),
      in_specs=[
          pl.BlockSpec((NC, BM, 64), lambda i: (0, i, 0)),
          pl.BlockSpec((64, 128), lambda i: (0, 0)),
          pl.BlockSpec((BM, 128), lambda i: (i, 0)),
          pl.BlockSpec((128, 128), lambda i: (0, 0)),
          pl.BlockSpec((1, 128), lambda i: (0, 0)),
      ],
      out_specs=[
          pl.BlockSpec((BM, 128), lambda i: (i, 0)),
          pl.BlockSpec((BM, 128), lambda i: (i, 0)),
      ],
      out_shape=[
          jax.ShapeDtypeStruct((N_NODES, 128), jnp.float32),
          jax.ShapeDtypeStruct((N_NODES, 128), jnp.float32),
      ],
  )(agg, wr2, root2, wo3, b3)


def _tc4_body(agg_ref, wr3_ref, root3_ref, batch_ref, wl_ref, bl_ref,
              out_ref, pooled_acc, counts_acc):
  i = pl.program_id(0)

  @pl.when(i == 0)
  def _():
    pooled_acc[...] = jnp.zeros_like(pooled_acc)
    counts_acc[...] = jnp.zeros_like(counts_acc)

  h3 = jnp.maximum(_dot(agg_ref[0] + agg_ref[1], wr3_ref[...])
                   + root3_ref[...], 0.0)
  b = batch_ref[0, 0, :]
  cols = lax.broadcasted_iota(jnp.int32, (BM, NUM_GRAPHS), 1)
  p = (b[:, None] == cols).astype(jnp.float32)
  pooled_acc[...] += lax.dot_general(p, h3, (((0,), (0,)), ((), ())),
                                     preferred_element_type=jnp.float32)
  counts_acc[...] += lax.dot_general(
      p, jnp.ones((BM, NUM_GRAPHS), jnp.float32),
      (((0,), (0,)), ((), ())), preferred_element_type=jnp.float32)

  @pl.when(i == GRID - 1)
  def _():
    pooled = pooled_acc[...] / jnp.maximum(counts_acc[...], 1.0)
    out_ref[...] = _dot(pooled, wl_ref[...]) + bl_ref[...]


def _tc4(agg, wr3, root3, batch3d, wl, bl):
  return pl.pallas_call(
      _tc4_body,
      grid=(GRID,),
      in_specs=[
          pl.BlockSpec((NC, BM, 128), lambda i: (0, i, 0)),
          pl.BlockSpec((128, 128), lambda i: (0, 0)),
          pl.BlockSpec((BM, 128), lambda i: (i, 0)),
          pl.BlockSpec((1, 1, BM), lambda i: (i, 0, 0)),
          pl.BlockSpec((128, 10), lambda i: (0, 0)),
          pl.BlockSpec((1, 10), lambda i: (0, 0)),
      ],
      out_specs=pl.BlockSpec((NUM_GRAPHS, 10), lambda i: (0, 0)),
      out_shape=jax.ShapeDtypeStruct((NUM_GRAPHS, 10), jnp.float32),
      scratch_shapes=[
          pltpu.VMEM((NUM_GRAPHS, 128), jnp.float32),
          pltpu.VMEM((NUM_GRAPHS, NUM_GRAPHS), jnp.float32),
      ],
  )(agg, wr3, root3, batch3d, wl, bl)


def kernel(x, edge_index, batch, W_stage1, b_stage1, W_rel1, W_root1, b1,
           W_rel2, W_root2, b2, W_rel3, W_root3, b3, W_lin, b_lin):
  src = edge_index[0]
  dst = edge_index[1]
  pad = E_PAD - N_EDGES
  srcs = jnp.concatenate(
      [src, jnp.zeros((pad,), jnp.int32)]).reshape(NW, C_T, K)
  dummy_dst = N_NODES + (jnp.arange(pad, dtype=jnp.int32) % (N_PAD - N_NODES))
  dsts = jnp.concatenate([dst, dummy_dst]).reshape(NW, C_T, K)
  z64 = jnp.zeros((R, 64), jnp.float32)
  z128 = jnp.zeros((R, 128), jnp.float32)
  batch3d = batch.reshape(GRID, 1, BM)

  bs = b_stage1.reshape(1, -1)
  b1r = b1.reshape(1, -1)
  b2r = b2.reshape(1, -1)
  b3r = b3.reshape(1, -1)
  blr = b_lin.reshape(1, -1)

  table1, root1 = _tc1(x, W_stage1, bs, W_rel1, W_root1, b1r)
  agg1 = _sc_segsum64(table1, srcs, dsts, z64)
  table2, root2 = _tc2(agg1, root1, W_root2, b2r)
  agg2 = _sc_segsum64(table2, srcs, dsts, z64)
  table3, root3 = _tc3(agg2, W_rel2, root2, W_root3, b3r)
  agg3 = _sc_segsum128(table3, srcs, dsts, z128)
  out = _tc4(agg3, W_rel3, root3, batch3d, W_lin, blr)
  return out


# speedup vs baseline: 1.2197x; 1.2197x over previous
import functools

import numpy as np

import jax
import jax.numpy as jnp
from jax import lax
from jax.experimental import pallas as pl
from jax.experimental.pallas import tpu as pltpu, tpu_sc as plsc

N_NODES = 10000
N_EDGES = 320000
NUM_GRAPHS = 128

NC = 2
NS = 16
NW = NC * NS
K = 128
CH = N_EDGES // K
CF = 98
CS = 59
CH_PAD = NS * (CF + CS)
E_PAD = CH_PAD * K
N_PAD = 10112
R = N_PAD // NS
DUMMY_ROW = N_NODES + 1

BM = 1000
GRID = N_NODES // BM


def _make_sc_segsum(D):
  mesh = plsc.VectorSubcoreMesh(core_axis_name="c", subcore_axis_name="s")

  @functools.partial(
      pl.kernel,
      out_type=jax.ShapeDtypeStruct((NC, N_PAD, D), jnp.float32),
      mesh=mesh,
      compiler_params=pltpu.CompilerParams(use_tc_tiling_on_sc=False),
      scratch_types=[
          pltpu.VMEM((CF, K), jnp.int32),
          pltpu.VMEM((CF, K), jnp.int32),
          pltpu.VMEM((K, D), jnp.float32),
          pltpu.VMEM_SHARED((N_PAD, D), jnp.float32),
          pltpu.SemaphoreType.DMA,
      ],
  )
  def seg(table_hbm, srcs_hbm, dsts_hbm, zeros_hbm, out_hbm,
          src_v, dst_v, rows_v, acc_sh, sem):
    c = lax.axis_index("c")
    s = lax.axis_index("s")
    wid = s * NC + c
    cnt = jnp.where(c == 0, CF, CS)

    pltpu.sync_copy(srcs_hbm.at[wid], src_v)
    pltpu.sync_copy(dsts_hbm.at[wid], dst_v)
    pltpu.sync_copy(zeros_hbm, acc_sh.at[pl.ds(s * R, R)])
    plsc.subcore_barrier()

    def body(j, carry):
      pltpu.async_copy(table_hbm.at[src_v.at[j]], rows_v, sem).wait()
      pltpu.sync_copy(rows_v, acc_sh.at[dst_v.at[j]], add=True)
      return carry

    lax.fori_loop(0, cnt, body, 0)
    plsc.subcore_barrier()
    pltpu.sync_copy(acc_sh.at[pl.ds(s * R, R)],
                    out_hbm.at[c, pl.ds(s * R, R)])

  return seg


_sc_segsum64 = _make_sc_segsum(64)
_sc_segsum128 = _make_sc_segsum(128)


def _dot(a, b):
  return lax.dot_general(a, b, (((1,), (0,)), ((), ())),
                         preferred_element_type=jnp.float32)


def _tc1_body(x_ref, ws_ref, bs_ref, wr_ref, wo_ref, b1_ref,
              table_ref, root_ref):
  h0 = _dot(x_ref[...], ws_ref[...]) + bs_ref[...]
  table_ref[...] = _dot(h0, wr_ref[...])
  root_ref[...] = _dot(h0, wo_ref[...]) + b1_ref[...]


def _tc1(x, ws, bs, wr, wo, b1):
  return pl.pallas_call(
      _tc1_body,
      grid=(GRID,),
      in_specs=[
          pl.BlockSpec((BM, 128), lambda i: (i, 0)),
          pl.BlockSpec((128, 128), lambda i: (0, 0)),
          pl.BlockSpec((1, 128), lambda i: (0, 0)),
          pl.BlockSpec((128, 64), lambda i: (0, 0)),
          pl.BlockSpec((128, 64), lambda i: (0, 0)),
          pl.BlockSpec((1, 64), lambda i: (0, 0)),
      ],
      out_specs=[
          pl.BlockSpec((BM, 64), lambda i: (i, 0)),
          pl.BlockSpec((BM, 64), lambda i: (i, 0)),
      ],
      out_shape=[
          jax.ShapeDtypeStruct((N_NODES, 64), jnp.float32),
          jax.ShapeDtypeStruct((N_NODES, 64), jnp.float32),
      ],
  )(x, ws, bs, wr, wo, b1)


def _tc2_body(agg_ref, root1_ref, wo2_ref, b2_ref, table_ref, root2_ref):
  h1 = jnp.maximum(agg_ref[0] + agg_ref[1] + root1_ref[...], 0.0)
  table_ref[...] = h1
  root2_ref[...] = _dot(h1, wo2_ref[...]) + b2_ref[...]


def _tc2(agg, root1, wo2, b2):
  return pl.pallas_call(
      _tc2_body,
      grid=(GRID,),
      in_specs=[
          pl.BlockSpec((NC, BM, 64), lambda i: (0, i, 0)),
          pl.BlockSpec((BM, 64), lambda i: (i, 0)),
          pl.BlockSpec((64, 128), lambda i: (0, 0)),
          pl.BlockSpec((1, 128), lambda i: (0, 0)),
      ],
      out_specs=[
          pl.BlockSpec((BM, 64), lambda i: (i, 0)),
          pl.BlockSpec((BM, 128), lambda i: (i, 0)),
      ],
      out_shape=[
          jax.ShapeDtypeStruct((N_NODES, 64), jnp.float32),
          jax.ShapeDtypeStruct((N_NODES, 128), jnp.float32),
      ],
  )(agg, root1, wo2, b2)


def _tc3_body(agg_ref, wr2_ref, root2_ref, wo3_ref, b3_ref,
              table_ref, root3_ref):
  h2 = jnp.maximum(_dot(agg_ref[0] + agg_ref[1], wr2_ref[...])
                   + root2_ref[...], 0.0)
  table_ref[...] = h2
  root3_ref[...] = _dot(h2, wo3_ref[...]) + b3_ref[...]


def _tc3(agg, wr2, root2, wo3, b3):
  return pl.pallas_call(
      _tc3_body,
      grid=(GRID,),
      in_specs=[
          pl.BlockSpec((NC, BM, 64), lambda i: (0, i, 0)),
          pl.BlockSpec((64, 128), lambda i: (0, 0)),
          pl.BlockSpec((BM, 128), lambda i: (i, 0)),
          pl.BlockSpec((128, 128), lambda i: (0, 0)),
          pl.BlockSpec((1, 128), lambda i: (0, 0)),
      ],
      out_specs=[
          pl.BlockSpec((BM, 128), lambda i: (i, 0)),
          pl.BlockSpec((BM, 128), lambda i: (i, 0)),
      ],
      out_shape=[
          jax.ShapeDtypeStruct((N_NODES, 128), jnp.float32),
          jax.ShapeDtypeStruct((N_NODES, 128), jnp.float32),
      ],
  )(agg, wr2, root2, wo3, b3)


def _tc4_body(agg_ref, wr3_ref, root3_ref, batch_ref, wl_ref, bl_ref,
              out_ref, pooled_acc, counts_acc):
  i = pl.program_id(0)

  @pl.when(i == 0)
  def _():
    pooled_acc[...] = jnp.zeros_like(pooled_acc)
    counts_acc[...] = jnp.zeros_like(counts_acc)

  h3 = jnp.maximum(_dot(agg_ref[0] + agg_ref[1], wr3_ref[...])
                   + root3_ref[...], 0.0)
  b = batch_ref[0, 0, :]
  cols = lax.broadcasted_iota(jnp.int32, (BM, NUM_GRAPHS), 1)
  p = (b[:, None] == cols).astype(jnp.float32)
  pooled_acc[...] += lax.dot_general(p, h3, (((0,), (0,)), ((), ())),
                                     preferred_element_type=jnp.float32)
  counts_acc[...] += lax.dot_general(
      p, jnp.ones((BM, NUM_GRAPHS), jnp.float32),
      (((0,), (0,)), ((), ())), preferred_element_type=jnp.float32)

  @pl.when(i == GRID - 1)
  def _():
    pooled = pooled_acc[...] / jnp.maximum(counts_acc[...], 1.0)
    out_ref[...] = _dot(pooled, wl_ref[...]) + bl_ref[...]


def _tc4(agg, wr3, root3, batch3d, wl, bl):
  return pl.pallas_call(
      _tc4_body,
      grid=(GRID,),
      in_specs=[
          pl.BlockSpec((NC, BM, 128), lambda i: (0, i, 0)),
          pl.BlockSpec((128, 128), lambda i: (0, 0)),
          pl.BlockSpec((BM, 128), lambda i: (i, 0)),
          pl.BlockSpec((1, 1, BM), lambda i: (i, 0, 0)),
          pl.BlockSpec((128, 10), lambda i: (0, 0)),
          pl.BlockSpec((1, 10), lambda i: (0, 0)),
      ],
      out_specs=pl.BlockSpec((NUM_GRAPHS, 10), lambda i: (0, 0)),
      out_shape=jax.ShapeDtypeStruct((NUM_GRAPHS, 10), jnp.float32),
      scratch_shapes=[
          pltpu.VMEM((NUM_GRAPHS, 128), jnp.float32),
          pltpu.VMEM((NUM_GRAPHS, NUM_GRAPHS), jnp.float32),
      ],
  )(agg, wr3, root3, batch3d, wl, bl)


def kernel(x, edge_index, batch, W_stage1, b_stage1, W_rel1, W_root1, b1,
           W_rel2, W_root2, b2, W_rel3, W_root3, b3, W_lin, b_lin):
  src = edge_index[0]
  dst = edge_index[1]
  pad = E_PAD - N_EDGES
  src_ch = jnp.concatenate(
      [src, jnp.zeros((pad,), jnp.int32)]).reshape(CH_PAD, K)
  dummy_dst = N_NODES + (jnp.arange(pad, dtype=jnp.int32) % (N_PAD - N_NODES))
  dst_ch = jnp.concatenate([dst, dummy_dst]).reshape(CH_PAD, K)
  cidx = np.full((NW, CF), CH_PAD - 1, np.int32)
  for t in range(NS):
    cidx[t * NC + 0, :] = np.arange(t * CF, (t + 1) * CF)
    cidx[t * NC + 1, :CS] = NS * CF + np.arange(t * CS, (t + 1) * CS)
  cidx = jnp.asarray(cidx)
  srcs = jnp.take(src_ch, cidx, axis=0)
  dsts = jnp.take(dst_ch, cidx, axis=0)
  z64 = jnp.zeros((R, 64), jnp.float32)
  z128 = jnp.zeros((R, 128), jnp.float32)
  batch3d = batch.reshape(GRID, 1, BM)

  bs = b_stage1.reshape(1, -1)
  b1r = b1.reshape(1, -1)
  b2r = b2.reshape(1, -1)
  b3r = b3.reshape(1, -1)
  blr = b_lin.reshape(1, -1)

  table1, root1 = _tc1(x, W_stage1, bs, W_rel1, W_root1, b1r)
  agg1 = _sc_segsum64(table1, srcs, dsts, z64)
  table2, root2 = _tc2(agg1, root1, W_root2, b2r)
  agg2 = _sc_segsum64(table2, srcs, dsts, z64)
  table3, root3 = _tc3(agg2, W_rel2, root2, W_root3, b3r)
  agg3 = _sc_segsum128(table3, srcs, dsts, z128)
  out = _tc4(agg3, W_rel3, root3, batch3d, W_lin, blr)
  return out


# speedup vs baseline: 1.3021x; 1.0676x over previous
import functools

import numpy as np

import jax
import jax.numpy as jnp
from jax import lax
from jax.experimental import pallas as pl
from jax.experimental.pallas import tpu as pltpu, tpu_sc as plsc

N_NODES = 10000
N_EDGES = 320000
NUM_GRAPHS = 128

NC = 2
NS = 16
NW = NC * NS
K = 128
CH = N_EDGES // K
CF = 92
CS = 65
CH_PAD = NS * (CF + CS)
E_PAD = CH_PAD * K
N_PAD = 10112
R = N_PAD // NS
DUMMY_ROW = N_NODES + 1

BM = 1000
GRID = N_NODES // BM


def _make_sc_segsum(D):
  mesh = plsc.VectorSubcoreMesh(core_axis_name="c", subcore_axis_name="s")

  @functools.partial(
      pl.kernel,
      out_type=jax.ShapeDtypeStruct((NC, N_PAD, D), jnp.float32),
      mesh=mesh,
      compiler_params=pltpu.CompilerParams(use_tc_tiling_on_sc=False),
      scratch_types=[
          pltpu.VMEM((CF, K), jnp.int32),
          pltpu.VMEM((CF, K), jnp.int32),
          pltpu.VMEM((K, D), jnp.float32),
          pltpu.VMEM_SHARED((N_PAD, D), jnp.float32),
          pltpu.SemaphoreType.DMA,
      ],
  )
  def seg(table_hbm, srcs_hbm, dsts_hbm, zeros_hbm, out_hbm,
          src_v, dst_v, rows_v, acc_sh, sem):
    c = lax.axis_index("c")
    s = lax.axis_index("s")
    wid = s * NC + c
    cnt = jnp.where(c == 0, CF, CS)

    pltpu.sync_copy(srcs_hbm.at[wid], src_v)
    pltpu.sync_copy(dsts_hbm.at[wid], dst_v)
    pltpu.sync_copy(zeros_hbm, acc_sh.at[pl.ds(s * R, R)])
    plsc.subcore_barrier()

    def body(j, carry):
      pltpu.async_copy(table_hbm.at[src_v.at[j]], rows_v, sem).wait()
      pltpu.sync_copy(rows_v, acc_sh.at[dst_v.at[j]], add=True)
      return carry

    lax.fori_loop(0, cnt, body, 0)
    plsc.subcore_barrier()
    pltpu.sync_copy(acc_sh.at[pl.ds(s * R, R)],
                    out_hbm.at[c, pl.ds(s * R, R)])

  return seg


_sc_segsum64 = _make_sc_segsum(64)
_sc_segsum128 = _make_sc_segsum(128)


def _dot(a, b):
  return lax.dot_general(a, b, (((1,), (0,)), ((), ())),
                         preferred_element_type=jnp.float32)


def _tc1_body(x_ref, ws_ref, bs_ref, wr_ref, wo_ref, b1_ref,
              table_ref, root_ref):
  h0 = _dot(x_ref[...], ws_ref[...]) + bs_ref[...]
  table_ref[...] = _dot(h0, wr_ref[...])
  root_ref[...] = _dot(h0, wo_ref[...]) + b1_ref[...]


def _tc1(x, ws, bs, wr, wo, b1):
  return pl.pallas_call(
      _tc1_body,
      grid=(GRID,),
      in_specs=[
          pl.BlockSpec((BM, 128), lambda i: (i, 0)),
          pl.BlockSpec((128, 128), lambda i: (0, 0)),
          pl.BlockSpec((1, 128), lambda i: (0, 0)),
          pl.BlockSpec((128, 64), lambda i: (0, 0)),
          pl.BlockSpec((128, 64), lambda i: (0, 0)),
          pl.BlockSpec((1, 64), lambda i: (0, 0)),
      ],
      out_specs=[
          pl.BlockSpec((BM, 64), lambda i: (i, 0)),
          pl.BlockSpec((BM, 64), lambda i: (i, 0)),
      ],
      out_shape=[
          jax.ShapeDtypeStruct((N_NODES, 64), jnp.float32),
          jax.ShapeDtypeStruct((N_NODES, 64), jnp.float32),
      ],
  )(x, ws, bs, wr, wo, b1)


def _tc2_body(agg_ref, root1_ref, wo2_ref, b2_ref, table_ref, root2_ref):
  h1 = jnp.maximum(agg_ref[0] + agg_ref[1] + root1_ref[...], 0.0)
  table_ref[...] = h1
  root2_ref[...] = _dot(h1, wo2_ref[...]) + b2_ref[...]


def _tc2(agg, root1, wo2, b2):
  return pl.pallas_call(
      _tc2_body,
      grid=(GRID,),
      in_specs=[
          pl.BlockSpec((NC, BM, 64), lambda i: (0, i, 0)),
          pl.BlockSpec((BM, 64), lambda i: (i, 0)),
          pl.BlockSpec((64, 128), lambda i: (0, 0)),
          pl.BlockSpec((1, 128), lambda i: (0, 0)),
      ],
      out_specs=[
          pl.BlockSpec((BM, 64), lambda i: (i, 0)),
          pl.BlockSpec((BM, 128), lambda i: (i, 0)),
      ],
      out_shape=[
          jax.ShapeDtypeStruct((N_NODES, 64), jnp.float32),
          jax.ShapeDtypeStruct((N_NODES, 128), jnp.float32),
      ],
  )(agg, root1, wo2, b2)


def _tc3_body(agg_ref, wr2_ref, root2_ref, wo3_ref, b3_ref,
              table_ref, root3_ref):
  h2 = jnp.maximum(_dot(agg_ref[0] + agg_ref[1], wr2_ref[...])
                   + root2_ref[...], 0.0)
  table_ref[...] = h2
  root3_ref[...] = _dot(h2, wo3_ref[...]) + b3_ref[...]


def _tc3(agg, wr2, root2, wo3, b3):
  return pl.pallas_call(
      _tc3_body,
      grid=(GRID,),
      in_specs=[
          pl.BlockSpec((NC, BM, 64), lambda i: (0, i, 0)),
          pl.BlockSpec((64, 128), lambda i: (0, 0)),
          pl.BlockSpec((BM, 128), lambda i: (i, 0)),
          pl.BlockSpec((128, 128), lambda i: (0, 0)),
          pl.BlockSpec((1, 128), lambda i: (0, 0)),
      ],
      out_specs=[
          pl.BlockSpec((BM, 128), lambda i: (i, 0)),
          pl.BlockSpec((BM, 128), lambda i: (i, 0)),
      ],
      out_shape=[
          jax.ShapeDtypeStruct((N_NODES, 128), jnp.float32),
          jax.ShapeDtypeStruct((N_NODES, 128), jnp.float32),
      ],
  )(agg, wr2, root2, wo3, b3)


def _tc4_body(agg_ref, wr3_ref, root3_ref, batch_ref, wl_ref, bl_ref,
              out_ref, pooled_acc, counts_acc):
  i = pl.program_id(0)

  @pl.when(i == 0)
  def _():
    pooled_acc[...] = jnp.zeros_like(pooled_acc)
    counts_acc[...] = jnp.zeros_like(counts_acc)

  h3 = jnp.maximum(_dot(agg_ref[0] + agg_ref[1], wr3_ref[...])
                   + root3_ref[...], 0.0)
  b = batch_ref[0, 0, :]
  cols = lax.broadcasted_iota(jnp.int32, (BM, NUM_GRAPHS), 1)
  p = (b[:, None] == cols).astype(jnp.float32)
  pooled_acc[...] += lax.dot_general(p, h3, (((0,), (0,)), ((), ())),
                                     preferred_element_type=jnp.float32)
  counts_acc[...] += lax.dot_general(
      p, jnp.ones((BM, NUM_GRAPHS), jnp.float32),
      (((0,), (0,)), ((), ())), preferred_element_type=jnp.float32)

  @pl.when(i == GRID - 1)
  def _():
    pooled = pooled_acc[...] / jnp.maximum(counts_acc[...], 1.0)
    out_ref[...] = _dot(pooled, wl_ref[...]) + bl_ref[...]


def _tc4(agg, wr3, root3, batch3d, wl, bl):
  return pl.pallas_call(
      _tc4_body,
      grid=(GRID,),
      in_specs=[
          pl.BlockSpec((NC, BM, 128), lambda i: (0, i, 0)),
          pl.BlockSpec((128, 128), lambda i: (0, 0)),
          pl.BlockSpec((BM, 128), lambda i: (i, 0)),
          pl.BlockSpec((1, 1, BM), lambda i: (i, 0, 0)),
          pl.BlockSpec((128, 10), lambda i: (0, 0)),
          pl.BlockSpec((1, 10), lambda i: (0, 0)),
      ],
      out_specs=pl.BlockSpec((NUM_GRAPHS, 10), lambda i: (0, 0)),
      out_shape=jax.ShapeDtypeStruct((NUM_GRAPHS, 10), jnp.float32),
      scratch_shapes=[
          pltpu.VMEM((NUM_GRAPHS, 128), jnp.float32),
          pltpu.VMEM((NUM_GRAPHS, NUM_GRAPHS), jnp.float32),
      ],
  )(agg, wr3, root3, batch3d, wl, bl)


def kernel(x, edge_index, batch, W_stage1, b_stage1, W_rel1, W_root1, b1,
           W_rel2, W_root2, b2, W_rel3, W_root3, b3, W_lin, b_lin):
  src = edge_index[0]
  dst = edge_index[1]
  pad = E_PAD - N_EDGES
  src_ch = jnp.concatenate(
      [src, jnp.zeros((pad,), jnp.int32)]).reshape(CH_PAD, K)
  dummy_dst = N_NODES + (jnp.arange(pad, dtype=jnp.int32) % (N_PAD - N_NODES))
  dst_ch = jnp.concatenate([dst, dummy_dst]).reshape(CH_PAD, K)
  cidx = np.full((NW, CF), CH_PAD - 1, np.int32)
  for t in range(NS):
    cidx[t * NC + 0, :] = np.arange(t * CF, (t + 1) * CF)
    cidx[t * NC + 1, :CS] = NS * CF + np.arange(t * CS, (t + 1) * CS)
  cidx = jnp.asarray(cidx)
  srcs = jnp.take(src_ch, cidx, axis=0)
  dsts = jnp.take(dst_ch, cidx, axis=0)
  z64 = jnp.zeros((R, 64), jnp.float32)
  z128 = jnp.zeros((R, 128), jnp.float32)
  batch3d = batch.reshape(GRID, 1, BM)

  bs = b_stage1.reshape(1, -1)
  b1r = b1.reshape(1, -1)
  b2r = b2.reshape(1, -1)
  b3r = b3.reshape(1, -1)
  blr = b_lin.reshape(1, -1)

  table1, root1 = _tc1(x, W_stage1, bs, W_rel1, W_root1, b1r)
  agg1 = _sc_segsum64(table1, srcs, dsts, z64)
  table2, root2 = _tc2(agg1, root1, W_root2, b2r)
  agg2 = _sc_segsum64(table2, srcs, dsts, z64)
  table3, root3 = _tc3(agg2, W_rel2, root2, W_root3, b3r)
  agg3 = _sc_segsum128(table3, srcs, dsts, z128)
  out = _tc4(agg3, W_rel3, root3, batch3d, W_lin, blr)
  return out


# speedup vs baseline: 1.3112x; 1.0070x over previous
import functools

import numpy as np

import jax
import jax.numpy as jnp
from jax import lax
from jax.experimental import pallas as pl
from jax.experimental.pallas import tpu as pltpu, tpu_sc as plsc

N_NODES = 10000
N_EDGES = 320000
NUM_GRAPHS = 128

NC = 2
NS = 16
NW = NC * NS
K = 128
CH = N_EDGES // K
CF = 90
CS = 67
CH_PAD = NS * (CF + CS)
E_PAD = CH_PAD * K
N_PAD = 10112
R = N_PAD // NS
DUMMY_ROW = N_NODES + 1

BM = 1000
GRID = N_NODES // BM


def _make_sc_segsum(D):
  mesh = plsc.VectorSubcoreMesh(core_axis_name="c", subcore_axis_name="s")

  @functools.partial(
      pl.kernel,
      out_type=jax.ShapeDtypeStruct((NC, N_PAD, D), jnp.float32),
      mesh=mesh,
      compiler_params=pltpu.CompilerParams(use_tc_tiling_on_sc=False),
      scratch_types=[
          pltpu.VMEM((CF, K), jnp.int32),
          pltpu.VMEM((CF, K), jnp.int32),
          pltpu.VMEM((K, D), jnp.float32),
          pltpu.VMEM_SHARED((N_PAD, D), jnp.float32),
          pltpu.SemaphoreType.DMA,
      ],
  )
  def seg(table_hbm, srcs_hbm, dsts_hbm, zeros_hbm, out_hbm,
          src_v, dst_v, rows_v, acc_sh, sem):
    c = lax.axis_index("c")
    s = lax.axis_index("s")
    wid = s * NC + c
    cnt = jnp.where(c == 0, CF, CS)

    pltpu.sync_copy(srcs_hbm.at[wid], src_v)
    pltpu.sync_copy(dsts_hbm.at[wid], dst_v)
    pltpu.sync_copy(zeros_hbm, acc_sh.at[pl.ds(s * R, R)])
    plsc.subcore_barrier()

    def body(j, carry):
      pltpu.async_copy(table_hbm.at[src_v.at[j]], rows_v, sem).wait()
      pltpu.sync_copy(rows_v, acc_sh.at[dst_v.at[j]], add=True)
      return carry

    lax.fori_loop(0, cnt, body, 0)
    plsc.subcore_barrier()
    pltpu.sync_copy(acc_sh.at[pl.ds(s * R, R)],
                    out_hbm.at[c, pl.ds(s * R, R)])

  return seg


_sc_segsum64 = _make_sc_segsum(64)
_sc_segsum128 = _make_sc_segsum(128)


def _dot(a, b):
  return lax.dot_general(a, b, (((1,), (0,)), ((), ())),
                         preferred_element_type=jnp.float32)


def _tc1_body(x_ref, ws_ref, bs_ref, wr_ref, wo_ref, b1_ref,
              table_ref, root_ref):
  h0 = _dot(x_ref[...], ws_ref[...]) + bs_ref[...]
  table_ref[...] = _dot(h0, wr_ref[...])
  root_ref[...] = _dot(h0, wo_ref[...]) + b1_ref[...]


def _tc1(x, ws, bs, wr, wo, b1):
  return pl.pallas_call(
      _tc1_body,
      grid=(GRID,),
      in_specs=[
          pl.BlockSpec((BM, 128), lambda i: (i, 0)),
          pl.BlockSpec((128, 128), lambda i: (0, 0)),
          pl.BlockSpec((1, 128), lambda i: (0, 0)),
          pl.BlockSpec((128, 64), lambda i: (0, 0)),
          pl.BlockSpec((128, 64), lambda i: (0, 0)),
          pl.BlockSpec((1, 64), lambda i: (0, 0)),
      ],
      out_specs=[
          pl.BlockSpec((BM, 64), lambda i: (i, 0)),
          pl.BlockSpec((BM, 64), lambda i: (i, 0)),
      ],
      out_shape=[
          jax.ShapeDtypeStruct((N_NODES, 64), jnp.float32),
          jax.ShapeDtypeStruct((N_NODES, 64), jnp.float32),
      ],
  )(x, ws, bs, wr, wo, b1)


def _tc2_body(agg_ref, root1_ref, wo2_ref, b2_ref, table_ref, root2_ref):
  h1 = jnp.maximum(agg_ref[0] + agg_ref[1] + root1_ref[...], 0.0)
  table_ref[...] = h1
  root2_ref[...] = _dot(h1, wo2_ref[...]) + b2_ref[...]


def _tc2(agg, root1, wo2, b2):
  return pl.pallas_call(
      _tc2_body,
      grid=(GRID,),
      in_specs=[
          pl.BlockSpec((NC, BM, 64), lambda i: (0, i, 0)),
          pl.BlockSpec((BM, 64), lambda i: (i, 0)),
          pl.BlockSpec((64, 128), lambda i: (0, 0)),
          pl.BlockSpec((1, 128), lambda i: (0, 0)),
      ],
      out_specs=[
          pl.BlockSpec((BM, 64), lambda i: (i, 0)),
          pl.BlockSpec((BM, 128), lambda i: (i, 0)),
      ],
      out_shape=[
          jax.ShapeDtypeStruct((N_NODES, 64), jnp.float32),
          jax.ShapeDtypeStruct((N_NODES, 128), jnp.float32),
      ],
  )(agg, root1, wo2, b2)


def _tc3_body(agg_ref, wr2_ref, root2_ref, wo3_ref, b3_ref,
              table_ref, root3_ref):
  h2 = jnp.maximum(_dot(agg_ref[0] + agg_ref[1], wr2_ref[...])
                   + root2_ref[...], 0.0)
  table_ref[...] = h2
  root3_ref[...] = _dot(h2, wo3_ref[...]) + b3_ref[...]


def _tc3(agg, wr2, root2, wo3, b3):
  return pl.pallas_call(
      _tc3_body,
      grid=(GRID,),
      in_specs=[
          pl.BlockSpec((NC, BM, 64), lambda i: (0, i, 0)),
          pl.BlockSpec((64, 128), lambda i: (0, 0)),
          pl.BlockSpec((BM, 128), lambda i: (i, 0)),
          pl.BlockSpec((128, 128), lambda i: (0, 0)),
          pl.BlockSpec((1, 128), lambda i: (0, 0)),
      ],
      out_specs=[
          pl.BlockSpec((BM, 128), lambda i: (i, 0)),
          pl.BlockSpec((BM, 128), lambda i: (i, 0)),
      ],
      out_shape=[
          jax.ShapeDtypeStruct((N_NODES, 128), jnp.float32),
          jax.ShapeDtypeStruct((N_NODES, 128), jnp.float32),
      ],
  )(agg, wr2, root2, wo3, b3)


def _tc4_body(agg_ref, wr3_ref, root3_ref, batch_ref, wl_ref, bl_ref,
              out_ref, pooled_acc, counts_acc):
  i = pl.program_id(0)

  @pl.when(i == 0)
  def _():
    pooled_acc[...] = jnp.zeros_like(pooled_acc)
    counts_acc[...] = jnp.zeros_like(counts_acc)

  h3 = jnp.maximum(_dot(agg_ref[0] + agg_ref[1], wr3_ref[...])
                   + root3_ref[...], 0.0)
  b = batch_ref[0, 0, :]
  cols = lax.broadcasted_iota(jnp.int32, (BM, NUM_GRAPHS), 1)
  p = (b[:, None] == cols).astype(jnp.float32)
  pooled_acc[...] += lax.dot_general(p, h3, (((0,), (0,)), ((), ())),
                                     preferred_element_type=jnp.float32)
  counts_acc[...] += lax.dot_general(
      p, jnp.ones((BM, NUM_GRAPHS), jnp.float32),
      (((0,), (0,)), ((), ())), preferred_element_type=jnp.float32)

  @pl.when(i == GRID - 1)
  def _():
    pooled = pooled_acc[...] / jnp.maximum(counts_acc[...], 1.0)
    out_ref[...] = _dot(pooled, wl_ref[...]) + bl_ref[...]


def _tc4(agg, wr3, root3, batch3d, wl, bl):
  return pl.pallas_call(
      _tc4_body,
      grid=(GRID,),
      in_specs=[
          pl.BlockSpec((NC, BM, 128), lambda i: (0, i, 0)),
          pl.BlockSpec((128, 128), lambda i: (0, 0)),
          pl.BlockSpec((BM, 128), lambda i: (i, 0)),
          pl.BlockSpec((1, 1, BM), lambda i: (i, 0, 0)),
          pl.BlockSpec((128, 10), lambda i: (0, 0)),
          pl.BlockSpec((1, 10), lambda i: (0, 0)),
      ],
      out_specs=pl.BlockSpec((NUM_GRAPHS, 10), lambda i: (0, 0)),
      out_shape=jax.ShapeDtypeStruct((NUM_GRAPHS, 10), jnp.float32),
      scratch_shapes=[
          pltpu.VMEM((NUM_GRAPHS, 128), jnp.float32),
          pltpu.VMEM((NUM_GRAPHS, NUM_GRAPHS), jnp.float32),
      ],
  )(agg, wr3, root3, batch3d, wl, bl)


def kernel(x, edge_index, batch, W_stage1, b_stage1, W_rel1, W_root1, b1,
           W_rel2, W_root2, b2, W_rel3, W_root3, b3, W_lin, b_lin):
  src = edge_index[0]
  dst = edge_index[1]
  pad = E_PAD - N_EDGES
  src_ch = jnp.concatenate(
      [src, jnp.zeros((pad,), jnp.int32)]).reshape(CH_PAD, K)
  dummy_dst = N_NODES + (jnp.arange(pad, dtype=jnp.int32) % (N_PAD - N_NODES))
  dst_ch = jnp.concatenate([dst, dummy_dst]).reshape(CH_PAD, K)
  cidx = np.full((NW, CF), CH_PAD - 1, np.int32)
  for t in range(NS):
    cidx[t * NC + 0, :] = np.arange(t * CF, (t + 1) * CF)
    cidx[t * NC + 1, :CS] = NS * CF + np.arange(t * CS, (t + 1) * CS)
  cidx = jnp.asarray(cidx)
  srcs = jnp.take(src_ch, cidx, axis=0)
  dsts = jnp.take(dst_ch, cidx, axis=0)
  z64 = jnp.zeros((R, 64), jnp.float32)
  z128 = jnp.zeros((R, 128), jnp.float32)
  batch3d = batch.reshape(GRID, 1, BM)

  bs = b_stage1.reshape(1, -1)
  b1r = b1.reshape(1, -1)
  b2r = b2.reshape(1, -1)
  b3r = b3.reshape(1, -1)
  blr = b_lin.reshape(1, -1)

  table1, root1 = _tc1(x, W_stage1, bs, W_rel1, W_root1, b1r)
  agg1 = _sc_segsum64(table1, srcs, dsts, z64)
  table2, root2 = _tc2(agg1, root1, W_root2, b2r)
  agg2 = _sc_segsum64(table2, srcs, dsts, z64)
  table3, root3 = _tc3(agg2, W_rel2, root2, W_root3, b3r)
  agg3 = _sc_segsum128(table3, srcs, dsts, z128)
  out = _tc4(agg3, W_rel3, root3, batch3d, W_lin, blr)
  return out
